# Initial kernel scaffold; baseline (speedup 1.0000x reference)
import jax
import jax.numpy as jnp
from jax.experimental import pallas as pl


def kernel(sentence, sentence_label, word_label, table, W, b):
    raise NotImplementedError("write your pallas kernel here")


# speedup vs baseline: 5.2461x; 5.2461x over previous
import jax
import jax.numpy as jnp
from jax import lax
from jax.experimental import pallas as pl
from jax.experimental.pallas import tpu as pltpu
from jax.experimental.pallas import tpu_sc as plsc

B = 16384
E = 64
VOCAB = 1000
TOK = 50
NIDX = B * TOK
G = TOK // 2

NC = 2
NS = 16
NW = NC * NS
PER_W = NIDX // NW
CH = 128
NCHUNK = PER_W // CH

N_PAD = 1024
BM = 512


def _sc_gather_body(idx_hbm, table_hbm, out_hbm, idx_v, rows_v, sem):
    wid = lax.axis_index("s") * NC + lax.axis_index("c")
    base = wid * PER_W
    pltpu.sync_copy(idx_hbm.at[wid], idx_v)

    def chunk(j, carry):
        pltpu.async_copy(table_hbm.at[idx_v.at[j]], rows_v, sem).wait()
        pltpu.sync_copy(rows_v, out_hbm.at[pl.ds(base + j * CH, CH)])
        return carry

    lax.fori_loop(0, NCHUNK, chunk, 0)


def _sc_gather(idx, table):
    mesh = plsc.VectorSubcoreMesh(core_axis_name="c", subcore_axis_name="s")
    return pl.kernel(
        _sc_gather_body,
        out_type=jax.ShapeDtypeStruct((NIDX, E), jnp.float32),
        mesh=mesh,
        compiler_params=pltpu.CompilerParams(use_tc_tiling_on_sc=False),
        scratch_types=[
            pltpu.VMEM((NCHUNK, CH), jnp.int32),
            pltpu.VMEM((CH, E), jnp.float32),
            pltpu.SemaphoreType.DMA,
        ],
    )(idx.reshape(NW, NCHUNK, CH), table)


def _mm_body(x_ref, w_ref, b_ref, o_ref):
    acc = jnp.broadcast_to(b_ref[0], (BM, N_PAD))
    for g in range(G):
        acc = acc + jnp.dot(
            x_ref[g], w_ref[g], preferred_element_type=jnp.float32
        )
    o_ref[...] = acc


def _tc_matmul(x3, w2, b_pad):
    return pl.pallas_call(
        _mm_body,
        grid=(B // BM,),
        in_specs=[
            pl.BlockSpec((G, BM, 2 * E), lambda m: (0, m, 0)),
            pl.BlockSpec((G, 2 * E, N_PAD), lambda m: (0, 0, 0)),
            pl.BlockSpec((1, N_PAD), lambda m: (0, 0)),
        ],
        out_specs=pl.BlockSpec((BM, N_PAD), lambda m: (m, 0)),
        out_shape=jax.ShapeDtypeStruct((B, N_PAD), jnp.float32),
    )(x3, w2, b_pad)


def kernel(sentence, sentence_label, word_label, table, W, b):
    idx = jnp.concatenate([sentence, sentence_label], axis=1)
    idx_pg = idx.reshape(B, G, 2).transpose(1, 0, 2).reshape(NIDX)
    gathered = _sc_gather(idx_pg, table)
    x3 = gathered.reshape(G, B, 2 * E)
    w_pad = jnp.pad(W, ((0, 0), (0, N_PAD - VOCAB)))
    w2 = w_pad.reshape(G, 2 * E, N_PAD)
    b_pad = jnp.pad(b, (0, N_PAD - VOCAB)).reshape(1, N_PAD)
    out = _tc_matmul(x3, w2, b_pad)
    return out[:, :VOCAB]


# speedup vs baseline: 7.2114x; 1.3746x over previous
import jax
import jax.numpy as jnp
from jax import lax
from jax.experimental import pallas as pl
from jax.experimental.pallas import tpu as pltpu
from jax.experimental.pallas import tpu_sc as plsc

B = 16384
E = 64
VOCAB = 1000
TOK = 50
NIDX = B * TOK
G = TOK // 2

NC = 2
NS = 16
NW = NC * NS
PER_W = NIDX // NW
CH = 128
NCHUNK = PER_W // CH
NSTR = 4
SUP = CH * NSTR
NSUP = PER_W // SUP
HSUP = NSUP // 2

BM = 1024


def _sc_gather_body(idx_hbm, table_hbm, out_hbm,
                    idx_v, rows0, rows1, sem_g0, sem_g1, sem_w0, sem_w1):
    wid = lax.axis_index("s") * NC + lax.axis_index("c")
    base = wid * PER_W
    pltpu.sync_copy(idx_hbm.at[wid], idx_v)
    rows = (rows0, rows1)
    sem_g = (sem_g0, sem_g1)
    sem_w = (sem_w0, sem_w1)

    def start_gathers(s, q):
        for p in range(NSTR):
            pltpu.make_async_copy(
                table_hbm.at[idx_v.at[s * NSTR + p]],
                rows[q].at[pl.ds(p * CH, CH)],
                sem_g[q],
            ).start()

    def wait_gathers(q):
        pltpu.make_async_copy(
            out_hbm.at[pl.ds(0, SUP)], rows[q], sem_g[q]
        ).wait()

    def wb(s, q):
        return pltpu.make_async_copy(
            rows[q], out_hbm.at[pl.ds(base + s * SUP, SUP)], sem_w[q]
        )

    start_gathers(0, 0)

    def body(ss, carry):
        s0 = ss * 2
        s1 = s0 + 1

        @pl.when(ss > 0)
        def _():
            wb(s0 - 1, 1).wait()

        start_gathers(s1, 1)
        wait_gathers(0)
        wb(s0, 0).start()
        wb(s0, 0).wait()

        @pl.when(ss < HSUP - 1)
        def _():
            start_gathers(s0 + 2, 0)

        wait_gathers(1)
        wb(s1, 1).start()
        return carry

    lax.fori_loop(0, HSUP, body, 0)
    wb(NSUP - 1, 1).wait()


def _sc_gather(idx, table):
    mesh = plsc.VectorSubcoreMesh(core_axis_name="c", subcore_axis_name="s")
    return pl.kernel(
        _sc_gather_body,
        out_type=jax.ShapeDtypeStruct((NIDX, E), jnp.float32),
        mesh=mesh,
        compiler_params=pltpu.CompilerParams(use_tc_tiling_on_sc=False),
        scratch_types=[
            pltpu.VMEM((NCHUNK, CH), jnp.int32),
            pltpu.VMEM((SUP, E), jnp.float32),
            pltpu.VMEM((SUP, E), jnp.float32),
            pltpu.SemaphoreType.DMA,
            pltpu.SemaphoreType.DMA,
            pltpu.SemaphoreType.DMA,
            pltpu.SemaphoreType.DMA,
        ],
    )(idx.reshape(NW, NCHUNK, CH), table)


def _mm_body(x_ref, w_ref, b_ref, o_ref, x2_ref):
    for g in range(G):
        x2_ref[:, pl.ds(g * 2 * E, 2 * E)] = x_ref[g].astype(jnp.bfloat16)
    o_ref[...] = (
        jnp.dot(x2_ref[...], w_ref[...], preferred_element_type=jnp.float32)
        + b_ref[...]
    )


def _tc_matmul(x3, w, b2):
    return pl.pallas_call(
        _mm_body,
        grid=(B // BM,),
        in_specs=[
            pl.BlockSpec((G, BM, 2 * E), lambda m: (0, m, 0)),
            pl.BlockSpec((TOK * E, VOCAB), lambda m: (0, 0)),
            pl.BlockSpec((1, VOCAB), lambda m: (0, 0)),
        ],
        out_specs=pl.BlockSpec((BM, VOCAB), lambda m: (m, 0)),
        out_shape=jax.ShapeDtypeStruct((B, VOCAB), jnp.float32),
        scratch_shapes=[pltpu.VMEM((BM, TOK * E), jnp.bfloat16)],
    )(x3, w, b2)


def kernel(sentence, sentence_label, word_label, table, W, b):
    idx = jnp.concatenate([sentence, sentence_label], axis=1)
    idx_pg = idx.reshape(B, G, 2).transpose(1, 0, 2).reshape(NIDX)
    gathered = _sc_gather(idx_pg, table)
    x3 = gathered.reshape(G, B, 2 * E)
    out = _tc_matmul(x3, W.astype(jnp.bfloat16), b.reshape(1, VOCAB))
    return out


# speedup vs baseline: 11.0334x; 1.5300x over previous
import jax
import jax.numpy as jnp
from jax import lax
from jax.experimental import pallas as pl
from jax.experimental.pallas import tpu as pltpu
from jax.experimental.pallas import tpu_sc as plsc

B = 16384
E = 64
VOCAB = 1000
TOK = 50
NIDX = B * TOK
G = TOK // 2

NC = 2
NS = 16
NW = NC * NS
PER_W = NIDX // NW
CH = 128
NCHUNK = PER_W // CH
NSTR = 4
SUP = CH * NSTR
NSUP = PER_W // SUP
HSUP = NSUP // 2

BM = 1024


BPW = B // NW
VPG = 2 * BPW // 16


def _sc_gather_body(idx_hbm, table_hbm, out_hbm,
                    idxb_v, idx_v, rows0, rows1,
                    sem_g0, sem_g1, sem_w0, sem_w1):
    wid = lax.axis_index("s") * NC + lax.axis_index("c")
    pltpu.sync_copy(idx_hbm.at[wid], idxb_v)

    it = lax.iota(jnp.int32, 16)
    patt = (it // 2) * TOK + (it % 2)
    for g in range(G):
        def permute(v2, carry, g=g):
            src = patt + (v2 * 8 * TOK + 2 * g)
            vec = plsc.load_gather(idxb_v, [src])
            idx_v[g * 8 + v2 // 8, pl.ds((v2 % 8) * 16, 16)] = vec
            return carry
        lax.fori_loop(0, VPG, permute, 0)

    rows = (rows0, rows1)
    sem_g = (sem_g0, sem_g1)
    sem_w = (sem_w0, sem_w1)

    def start_gathers(s, q):
        for p in range(NSTR):
            pltpu.make_async_copy(
                table_hbm.at[idx_v.at[s * NSTR + p]],
                rows[q].at[pl.ds(p * CH, CH)],
                sem_g[q],
            ).start()

    def wait_gathers(q):
        pltpu.make_async_copy(
            out_hbm.at[pl.ds(0, SUP)], rows[q], sem_g[q]
        ).wait()

    def out_base(s):
        return (s // 2) * (2 * B) + wid * (2 * BPW) + (s % 2) * SUP

    def wb(s, q):
        return pltpu.make_async_copy(
            rows[q], out_hbm.at[pl.ds(out_base(s), SUP)], sem_w[q]
        )

    start_gathers(0, 0)

    def body(ss, carry):
        s0 = ss * 2
        s1 = s0 + 1

        @pl.when(ss > 0)
        def _():
            wb(s0 - 1, 1).wait()

        start_gathers(s1, 1)
        wait_gathers(0)
        wb(s0, 0).start()
        wb(s0, 0).wait()

        @pl.when(ss < HSUP - 1)
        def _():
            start_gathers(s0 + 2, 0)

        wait_gathers(1)
        wb(s1, 1).start()
        return carry

    lax.fori_loop(0, HSUP, body, 0)
    wb(NSUP - 1, 1).wait()


def _sc_gather(conc, table):
    mesh = plsc.VectorSubcoreMesh(core_axis_name="c", subcore_axis_name="s")
    return pl.kernel(
        _sc_gather_body,
        out_type=jax.ShapeDtypeStruct((NIDX, E), jnp.float32),
        mesh=mesh,
        compiler_params=pltpu.CompilerParams(
            use_tc_tiling_on_sc=False, needs_layout_passes=False
        ),
        scratch_types=[
            pltpu.VMEM((PER_W,), jnp.int32),
            pltpu.VMEM((NCHUNK, CH), jnp.int32),
            pltpu.VMEM((SUP, E), jnp.float32),
            pltpu.VMEM((SUP, E), jnp.float32),
            pltpu.SemaphoreType.DMA,
            pltpu.SemaphoreType.DMA,
            pltpu.SemaphoreType.DMA,
            pltpu.SemaphoreType.DMA,
        ],
    )(conc.reshape(NW, PER_W), table)


def _mm_body(x_ref, w_ref, b_ref, o_ref, x2_ref):
    for g in range(G):
        x2_ref[:, pl.ds(g * 2 * E, 2 * E)] = x_ref[g].astype(jnp.bfloat16)
    o_ref[...] = (
        jnp.dot(x2_ref[...], w_ref[...], preferred_element_type=jnp.float32)
        + b_ref[...]
    )


def _tc_matmul(x3, w, b2):
    return pl.pallas_call(
        _mm_body,
        grid=(B // BM,),
        in_specs=[
            pl.BlockSpec((G, BM, 2 * E), lambda m: (0, m, 0)),
            pl.BlockSpec((TOK * E, VOCAB), lambda m: (0, 0)),
            pl.BlockSpec((1, VOCAB), lambda m: (0, 0)),
        ],
        out_specs=pl.BlockSpec((BM, VOCAB), lambda m: (m, 0)),
        out_shape=jax.ShapeDtypeStruct((B, VOCAB), jnp.float32),
        scratch_shapes=[pltpu.VMEM((BM, TOK * E), jnp.bfloat16)],
    )(x3, w, b2)


def kernel(sentence, sentence_label, word_label, table, W, b):
    conc = jnp.concatenate([sentence, sentence_label], axis=1)
    gathered = _sc_gather(conc, table)
    x3 = gathered.reshape(G, B, 2 * E)
    out = _tc_matmul(x3, W.astype(jnp.bfloat16), b.reshape(1, VOCAB))
    return out
